# baseline (speedup 1.0000x reference)
import functools

import jax
import jax.numpy as jnp
from jax import lax
from jax.experimental import pallas as pl
from jax.experimental.pallas import tpu as pltpu
from jax.experimental.pallas import tpu_sc as plsc

_NC = 2
_NS = 16
_L = 16

_N = 10000
_E = 320000
_D = 128
_DH = _D // _NC
_EPT = _E // _NS
_CHUNK = 128
_NCHUNK = -(-_EPT // _CHUNK)
_EPT_PAD = _NCHUNK * _CHUNK
_RPT = 640
_NPAD = _NS * _RPT
_NBUF = 3


def _pre_body(x_ref, w_ref, asrc_ref, adst_ref, h2_ref, as_ref, ad_ref):
    h = jnp.dot(x_ref[...], w_ref[...], preferred_element_type=jnp.float32)
    h2_ref[0] = h[:, :_DH].astype(jnp.bfloat16)
    h2_ref[1] = h[:, _DH:].astype(jnp.bfloat16)
    as_ref[...] = jnp.dot(h, asrc_ref[...], preferred_element_type=jnp.float32)
    ad_ref[...] = jnp.dot(h, adst_ref[...], preferred_element_type=jnp.float32)


def _post_body(n_ref, d_ref, b_ref, o_ref):
    rden = 1.0 / (d_ref[...] + 1e-16)

    def _fix(nh):
        blocks = []
        for g in range(_DH // 32):
            ev = nh[:, g * 32:g * 32 + 16]
            od = nh[:, g * 32 + 16:(g + 1) * 32]
            blocks.append(
                jnp.stack([ev, od], axis=2).reshape(nh.shape[0], 32))
        return jnp.concatenate(blocks, axis=1)

    num = jnp.concatenate([_fix(n_ref[0]), _fix(n_ref[1])], axis=1)
    o_ref[...] = jnp.maximum(num * rden + b_ref[...], 0.0)


def _sc_body(h2_h, as_h, ad_h, src3_h, dst3_h, num_h, den_h,
             asv, adv, srcv, dstv, exbufs, rows, sbuf,
             num_s, den_s, gsems, ssems):
    cid = lax.axis_index("c")
    sid = lax.axis_index("s")

    pltpu.sync_copy(as_h, asv)
    pltpu.sync_copy(ad_h, adv)
    pltpu.sync_copy(src3_h.at[sid], srcv)
    pltpu.sync_copy(dst3_h.at[sid], dstv)

    zeros16 = jnp.zeros((_L,), jnp.float32)
    htab = h2_h.at[cid]

    def _z0_body(j, c):
        for k in range(_DH // _L):
            sbuf[0, j, pl.ds(k * _L, _L)] = zeros16
        return c
    lax.fori_loop(0, _CHUNK, _z0_body, 0)
    for k in range(_CHUNK // _L):
        exbufs[0, pl.ds(k * _L, _L)] = zeros16

    base_row = sid * _RPT
    for i in range(_RPT // _CHUNK):
        off = base_row + i * _CHUNK
        pltpu.sync_copy(sbuf.at[0], num_s.at[pl.ds(off, _CHUNK)])
        pltpu.sync_copy(exbufs.at[0], den_s.at[pl.ds(off, _CHUNK)])

    plsc.subcore_barrier()

    def _start_gather(c, b):
        pltpu.async_copy(htab.at[srcv.at[c]], rows.at[b], gsems.at[b])

    def _wait_gather(b):
        pltpu.make_async_copy(h2_h.at[0, pl.ds(0, _CHUNK)], rows.at[b],
                              gsems.at[b]).wait()

    def _start_scatter(c, s):
        pltpu.async_copy(sbuf.at[s], num_s.at[dstv.at[c]], ssems.at[s],
                         add=True)
        pltpu.async_copy(exbufs.at[s], den_s.at[dstv.at[c]], ssems.at[s],
                         add=True)

    def _wait_scatter(s):
        pltpu.make_async_copy(sbuf.at[s], num_s.at[pl.ds(0, _CHUNK)],
                              ssems.at[s]).wait()
        pltpu.make_async_copy(exbufs.at[s], den_s.at[pl.ds(0, _CHUNK)],
                              ssems.at[s]).wait()

    lane = lax.iota(jnp.int32, _L)

    def _scale(c, b, s):
        def _sj(jg, cc):
            off = jg * _L
            si = srcv[c, pl.ds(off, _L)]
            di = dstv[c, pl.ds(off, _L)]
            e = plsc.load_gather(asv, [si]) + plsc.load_gather(adv, [di])
            e = jnp.where(e > 0.0, e, 0.2 * e)
            ex = jnp.exp(e)
            ex = jnp.where(c * _CHUNK + off + lane < _EPT, ex, 0.0)
            exbufs[s, pl.ds(off, _L)] = ex
            for j in range(_L):
                bs = jnp.full((_L,), ex[j], jnp.float32)
                row = off + j
                for g in range(_DH // 32):
                    v = rows[b, row, pl.ds(g * 32, 32)]
                    fe, fo = plsc.unpack(v, format=plsc.PackFormat.INTERLEAVED)
                    sbuf[s, row, pl.ds(g * 32, _L)] = fe * bs
                    sbuf[s, row, pl.ds(g * 32 + _L, _L)] = fo * bs
            return cc
        lax.fori_loop(0, _CHUNK // _L, _sj, 0)

    _start_gather(0, 0)
    _start_gather(1, 1)
    for c in range(2):
        _start_gather(c + 2, c + 2 if c + 2 < _NBUF else 0)
        _wait_gather(c)
        _scale(c, c, c % 2)
        _start_scatter(c, c % 2)

    def _slot(c):
        jb = lax.rem(c, _NBUF)
        sb = lax.rem(c, 2)

        @pl.when(c + 2 < _NCHUNK)
        def _():
            _start_gather(c + 2, lax.rem(c + 2, _NBUF))
        _wait_scatter(sb)
        _wait_gather(jb)
        _scale(c, jb, sb)
        _start_scatter(c, sb)

    def _main(c, carry):
        _slot(c)
        return carry
    lax.fori_loop(2, _NCHUNK, _main, 0)
    _wait_scatter(0)
    _wait_scatter(1)

    plsc.subcore_barrier()

    for i in range(_RPT // _CHUNK):
        off = base_row + i * _CHUNK
        pltpu.sync_copy(num_s.at[pl.ds(off, _CHUNK)], sbuf.at[0])
        pltpu.sync_copy(sbuf.at[0], num_h.at[cid, pl.ds(off, _CHUNK)])
        pltpu.sync_copy(den_s.at[pl.ds(off, _CHUNK)], exbufs.at[0])
        pltpu.sync_copy(exbufs.at[0],
                        den_h.at[pl.ds(cid * _NPAD + off, _CHUNK)])


@functools.cache
def _sc_kernel():
    mesh = plsc.VectorSubcoreMesh(core_axis_name="c", subcore_axis_name="s")
    return pl.kernel(
        _sc_body,
        out_type=[
            jax.ShapeDtypeStruct((_NC, _NPAD, _DH), jnp.float32),
            jax.ShapeDtypeStruct((_NC * _NPAD,), jnp.float32),
        ],
        mesh=mesh,
        compiler_params=pltpu.CompilerParams(
            needs_layout_passes=False, use_tc_tiling_on_sc=False),
        scratch_types=[
            pltpu.VMEM((_N,), jnp.float32),
            pltpu.VMEM((_N,), jnp.float32),
            pltpu.VMEM((_NCHUNK, _CHUNK), jnp.int32),
            pltpu.VMEM((_NCHUNK, _CHUNK), jnp.int32),
            pltpu.VMEM((2, _CHUNK), jnp.float32),
            pltpu.VMEM((_NBUF, _CHUNK, _DH), jnp.bfloat16),
            pltpu.VMEM((2, _CHUNK, _DH), jnp.float32),
            pltpu.VMEM_SHARED((_NPAD, _DH), jnp.float32),
            pltpu.VMEM_SHARED((_NPAD,), jnp.float32),
            pltpu.SemaphoreType.DMA((_NBUF,)),
            pltpu.SemaphoreType.DMA((2,)),
        ],
    )


def kernel(x, edge_index, W, att_src, att_dst, bias):
    blk = 1000
    grid = _N // blk
    h2, a_s, a_d = pl.pallas_call(
        _pre_body,
        grid=(grid,),
        in_specs=[
            pl.BlockSpec((blk, _D), lambda i: (i, 0)),
            pl.BlockSpec((_D, _D), lambda i: (0, 0)),
            pl.BlockSpec((_D, 1), lambda i: (0, 0)),
            pl.BlockSpec((_D, 1), lambda i: (0, 0)),
        ],
        out_specs=[
            pl.BlockSpec((2, blk, _DH), lambda i: (0, i, 0)),
            pl.BlockSpec((blk, 1), lambda i: (i, 0)),
            pl.BlockSpec((blk, 1), lambda i: (i, 0)),
        ],
        out_shape=[
            jax.ShapeDtypeStruct((2, _N, _DH), jnp.bfloat16),
            jax.ShapeDtypeStruct((_N, 1), jnp.float32),
            jax.ShapeDtypeStruct((_N, 1), jnp.float32),
        ],
    )(x, W, att_src[:, None], att_dst[:, None])

    src = edge_index[0].reshape(_NS, _EPT)
    dst = edge_index[1].reshape(_NS, _EPT)
    pad = _EPT_PAD - _EPT
    src3 = jnp.pad(src, ((0, 0), (0, pad))).reshape(_NS, _NCHUNK, _CHUNK)
    dst3 = jnp.pad(dst, ((0, 0), (0, pad))).reshape(_NS, _NCHUNK, _CHUNK)

    num, den = _sc_kernel()(h2, a_s.reshape(-1), a_d.reshape(-1), src3, dst3)
    den = den.reshape(_NC, _NPAD)

    out = pl.pallas_call(
        _post_body,
        grid=(grid,),
        in_specs=[
            pl.BlockSpec((2, blk, _DH), lambda i: (0, i, 0)),
            pl.BlockSpec((blk, 1), lambda i: (i, 0)),
            pl.BlockSpec((1, _D), lambda i: (0, 0)),
        ],
        out_specs=pl.BlockSpec((blk, _D), lambda i: (i, 0)),
        out_shape=jax.ShapeDtypeStruct((_N, _D), jnp.float32),
    )(num, den[0, :_N, None], bias[None, :])
    return out


# speedup vs baseline: 2.4575x; 2.4575x over previous
import functools

import jax
import jax.numpy as jnp
from jax import lax
from jax.experimental import pallas as pl
from jax.experimental.pallas import tpu as pltpu
from jax.experimental.pallas import tpu_sc as plsc

_NC = 2
_NS = 16
_L = 16

_N = 10000
_E = 320000
_D = 128
_DH = _D // _NC
_EPT = _E // _NS
_CHUNK = 128
_NCHUNK = -(-_EPT // _CHUNK)
_EPT_PAD = _NCHUNK * _CHUNK
_RPT = 640
_NPAD = _NS * _RPT
_NBUF = 3


def _pre_body(x_ref, w_ref, asrc_ref, adst_ref, h2_ref, as_ref, ad_ref):
    h = jnp.dot(x_ref[...], w_ref[...], preferred_element_type=jnp.float32)
    h2_ref[0] = h[:, :_DH]
    h2_ref[1] = h[:, _DH:]
    as_ref[...] = jnp.dot(h, asrc_ref[...], preferred_element_type=jnp.float32)
    ad_ref[...] = jnp.dot(h, adst_ref[...], preferred_element_type=jnp.float32)


def _sc_body(h2_h, as_h, ad_h, src3_h, dst3_h, bias_h, out_h,
             asv, adv, srcv, dstv, exbufs, rows, zb, bbuf,
             num_s, den_s, gsems, ssems):
    cid = lax.axis_index("c")
    sid = lax.axis_index("s")

    pltpu.sync_copy(as_h, asv)
    pltpu.sync_copy(ad_h, adv)
    pltpu.sync_copy(src3_h.at[sid], srcv)
    pltpu.sync_copy(dst3_h.at[sid], dstv)
    pltpu.sync_copy(bias_h.at[pl.ds(cid * _DH, _DH)], bbuf)

    zeros16 = jnp.zeros((_L,), jnp.float32)
    htab = h2_h.at[cid]

    def _zb_body(v, c):
        zb[pl.ds(v * _L, _L)] = zeros16
        return c
    lax.fori_loop(0, _RPT // _L, _zb_body, 0)

    def _r0_body(j, c):
        for k in range(_DH // _L):
            rows[0, j, pl.ds(k * _L, _L)] = zeros16
        return c
    lax.fori_loop(0, _CHUNK, _r0_body, 0)

    base_row = sid * _RPT
    for i in range(_RPT // _CHUNK):
        pltpu.sync_copy(rows.at[0], num_s.at[pl.ds(base_row + i * _CHUNK, _CHUNK)])
    pltpu.sync_copy(zb, den_s.at[pl.ds(base_row, _RPT)])

    plsc.subcore_barrier()

    def _start_gather(c, b):
        pltpu.async_copy(htab.at[srcv.at[c]], rows.at[b], gsems.at[b])

    def _wait_gather(b):
        pltpu.make_async_copy(h2_h.at[0, pl.ds(0, _CHUNK)], rows.at[b],
                              gsems.at[b]).wait()

    def _start_scatter(c, b):
        pltpu.async_copy(rows.at[b], num_s.at[dstv.at[c]], ssems.at[b],
                         add=True)
        pltpu.async_copy(exbufs.at[b], den_s.at[dstv.at[c]], ssems.at[b],
                         add=True)

    def _wait_scatter(b):
        pltpu.make_async_copy(rows.at[b], num_s.at[pl.ds(0, _CHUNK)],
                              ssems.at[b]).wait()
        pltpu.make_async_copy(exbufs.at[b], den_s.at[pl.ds(0, _CHUNK)],
                              ssems.at[b]).wait()

    lane = lax.iota(jnp.int32, _L)

    def _scale(c, b):
        def _sj(jg, cc):
            off = jg * _L
            si = srcv[c, pl.ds(off, _L)]
            di = dstv[c, pl.ds(off, _L)]
            e = plsc.load_gather(asv, [si]) + plsc.load_gather(adv, [di])
            e = jnp.where(e > 0.0, e, 0.2 * e)
            ex = jnp.exp(e)
            ex = jnp.where(c * _CHUNK + off + lane < _EPT, ex, 0.0)
            exbufs[b, pl.ds(off, _L)] = ex
            for j in range(_L):
                bs = jnp.full((_L,), ex[j], jnp.float32)
                row = off + j
                for k in range(_DH // _L):
                    rows[b, row, pl.ds(k * _L, _L)] = (
                        rows[b, row, pl.ds(k * _L, _L)] * bs)
            return cc
        lax.fori_loop(0, _CHUNK // _L, _sj, 0)

    for b in range(_NBUF):
        _start_gather(b, b)

    for j in range(_NBUF):
        if j == _NBUF - 1:
            _wait_scatter(0)
            _start_gather(_NBUF, 0)
        _wait_gather(j)
        _scale(j, j)
        _start_scatter(j, j)

    def _main(i, c):
        c0 = _NBUF * i
        for j in range(_NBUF):
            cj = c0 + j
            jn = (j + 1) % _NBUF
            _wait_scatter(jn)

            @pl.when(cj + 1 < _NCHUNK)
            def _():
                _start_gather(cj + 1, jn)
            _wait_gather(j)
            _scale(cj, j)
            _start_scatter(cj, j)
        return c
    lax.fori_loop(1, _NCHUNK // _NBUF, _main, 0)
    _wait_gather(0)
    _scale(_NCHUNK - 1, 0)
    _start_scatter(_NCHUNK - 1, 0)
    for b in range(_NBUF):
        _wait_scatter(b)

    plsc.subcore_barrier()

    for i in range(_RPT // _CHUNK):
        off = base_row + i * _CHUNK
        pltpu.sync_copy(num_s.at[pl.ds(off, _CHUNK)], rows.at[0])
        pltpu.sync_copy(den_s.at[pl.ds(off, _CHUNK)], exbufs.at[0])

        def _div(jg, cc):
            dvec = exbufs[0, pl.ds(jg * _L, _L)]
            rd = 1.0 / (dvec + 1e-16)
            for j in range(_L):
                rdj = jnp.full((_L,), rd[j], jnp.float32)
                row = jg * _L + j
                for k in range(_DH // _L):
                    v = rows[0, row, pl.ds(k * _L, _L)]
                    rows[0, row, pl.ds(k * _L, _L)] = jnp.maximum(
                        v * rdj + bbuf[pl.ds(k * _L, _L)], 0.0)
            return cc
        lax.fori_loop(0, _CHUNK // _L, _div, 0)

        @pl.when(off + _CHUNK <= _N)
        def _():
            pltpu.sync_copy(
                rows.at[0],
                out_h.at[pl.ds(off, _CHUNK), pl.ds(cid * _DH, _DH)])

        @pl.when(jnp.logical_and(off < _N, off + _CHUNK > _N))
        def _():
            pltpu.sync_copy(
                rows.at[0, pl.ds(0, _N % _CHUNK)],
                out_h.at[pl.ds(off, _N % _CHUNK), pl.ds(cid * _DH, _DH)])


@functools.cache
def _sc_kernel():
    mesh = plsc.VectorSubcoreMesh(core_axis_name="c", subcore_axis_name="s")
    return pl.kernel(
        _sc_body,
        out_type=jax.ShapeDtypeStruct((_N, _D), jnp.float32),
        mesh=mesh,
        compiler_params=pltpu.CompilerParams(
            needs_layout_passes=False, use_tc_tiling_on_sc=False),
        scratch_types=[
            pltpu.VMEM((_N,), jnp.float32),
            pltpu.VMEM((_N,), jnp.float32),
            pltpu.VMEM((_NCHUNK, _CHUNK), jnp.int32),
            pltpu.VMEM((_NCHUNK, _CHUNK), jnp.int32),
            pltpu.VMEM((_NBUF, _CHUNK), jnp.float32),
            pltpu.VMEM((_NBUF, _CHUNK, _DH), jnp.float32),
            pltpu.VMEM((_RPT,), jnp.float32),
            pltpu.VMEM((_DH,), jnp.float32),
            pltpu.VMEM_SHARED((_NPAD, _DH), jnp.float32),
            pltpu.VMEM_SHARED((_NPAD,), jnp.float32),
            pltpu.SemaphoreType.DMA((_NBUF,)),
            pltpu.SemaphoreType.DMA((_NBUF,)),
        ],
    )


def kernel(x, edge_index, W, att_src, att_dst, bias):
    blk = 1000
    grid = _N // blk
    h2, a_s, a_d = pl.pallas_call(
        _pre_body,
        grid=(grid,),
        in_specs=[
            pl.BlockSpec((blk, _D), lambda i: (i, 0)),
            pl.BlockSpec((_D, _D), lambda i: (0, 0)),
            pl.BlockSpec((_D, 1), lambda i: (0, 0)),
            pl.BlockSpec((_D, 1), lambda i: (0, 0)),
        ],
        out_specs=[
            pl.BlockSpec((2, blk, _DH), lambda i: (0, i, 0)),
            pl.BlockSpec((blk, 1), lambda i: (i, 0)),
            pl.BlockSpec((blk, 1), lambda i: (i, 0)),
        ],
        out_shape=[
            jax.ShapeDtypeStruct((2, _N, _DH), jnp.float32),
            jax.ShapeDtypeStruct((_N, 1), jnp.float32),
            jax.ShapeDtypeStruct((_N, 1), jnp.float32),
        ],
    )(x, W, att_src[:, None], att_dst[:, None])

    src = edge_index[0].reshape(_NS, _EPT)
    dst = edge_index[1].reshape(_NS, _EPT)
    pad = _EPT_PAD - _EPT
    src3 = jnp.pad(src, ((0, 0), (0, pad))).reshape(_NS, _NCHUNK, _CHUNK)
    dst3 = jnp.pad(dst, ((0, 0), (0, pad))).reshape(_NS, _NCHUNK, _CHUNK)

    return _sc_kernel()(h2, a_s.reshape(-1), a_d.reshape(-1), src3, dst3,
                        bias)
